# Initial kernel scaffold; baseline (speedup 1.0000x reference)
import jax
import jax.numpy as jnp
from jax.experimental import pallas as pl


def kernel(patch, params):
    raise NotImplementedError("write your pallas kernel here")


# speedup vs baseline: 3.6215x; 3.6215x over previous
import jax
import jax.numpy as jnp
from jax import lax
from jax.experimental import pallas as pl

EMBED_DIM = 64
NUM_EMB = 8192
H_DIM = 128
RES_H = 32

N_ROWS = 4 * 56 * 56
RB = 256
NB = N_ROWS // RB


def _conv(x, w, b, stride, padding):
    out = lax.conv_general_dilated(
        x, w, (stride, stride), [(padding, padding), (padding, padding)],
        dimension_numbers=('NCHW', 'OIHW', 'NCHW'))
    if b is not None:
        out = out + b[None, :, None, None]
    return out


def _conv_t(x, w, b, stride, padding):
    k = w.shape[2]
    pad = k - 1 - padding
    w2 = jnp.flip(w, (2, 3)).transpose(1, 0, 2, 3)
    out = lax.conv_general_dilated(
        x, w2, (1, 1), [(pad, pad), (pad, pad)], lhs_dilation=(stride, stride),
        dimension_numbers=('NCHW', 'OIHW', 'NCHW'))
    if b is not None:
        out = out + b[None, :, None, None]
    return out


def _res_stack(x, layers):
    for (w1, w2) in layers:
        h = _conv(jax.nn.relu(x), w1, None, 1, 1)
        h = _conv(jax.nn.relu(h), w2, None, 1, 0)
        x = x + h
    return jax.nn.relu(x)


def _lookup_body(idx_ref, emb_ref, out_ref):
    idxv = idx_ref[0, 0, :]
    cols = lax.broadcasted_iota(jnp.int32, (RB, NUM_EMB), 1)
    onehot = (idxv[:, None] == cols).astype(jnp.bfloat16)
    emb = emb_ref[...]
    eh = emb.astype(jnp.bfloat16)
    el = (emb - eh.astype(jnp.float32)).astype(jnp.bfloat16)
    dn = (((1,), (0,)), ((), ()))
    zq = (lax.dot_general(onehot, eh, dn, preferred_element_type=jnp.float32)
          + lax.dot_general(onehot, el, dn, preferred_element_type=jnp.float32))
    out_ref[...] = zq


_lookup_call = pl.pallas_call(
    _lookup_body,
    grid=(NB,),
    in_specs=[
        pl.BlockSpec((1, 1, RB), lambda i: (i, 0, 0)),
        pl.BlockSpec((NUM_EMB, EMBED_DIM), lambda i: (0, 0)),
    ],
    out_specs=pl.BlockSpec((RB, EMBED_DIM), lambda i: (i, 0)),
    out_shape=jax.ShapeDtypeStruct((N_ROWS, EMBED_DIM), jnp.float32),
)


def kernel(patch, params):
    z = jax.nn.relu(_conv(patch, params['enc_w1'], params['enc_b1'], 2, 1))
    z = jax.nn.relu(_conv(z, params['enc_w2'], params['enc_b2'], 2, 1))
    z = _conv(z, params['enc_w3'], params['enc_b3'], 1, 1)
    z = _res_stack(z, params['enc_res'])
    z = _conv(z, params['preq_w'], params['preq_b'], 1, 0)
    z = z.transpose(0, 2, 3, 1)
    zf = z.reshape(-1, EMBED_DIM)
    emb = params['emb']

    dists = (jnp.sum(zf ** 2, axis=1, keepdims=True) + jnp.sum(emb ** 2, axis=1)
             - 2.0 * (zf @ emb.T))
    idx = jnp.argmin(dists, axis=1)

    z_q = _lookup_call(idx.reshape(NB, 1, RB), emb).reshape(z.shape)

    commitment_loss = jnp.mean((lax.stop_gradient(z_q) - z) ** 2)
    codebook_loss = jnp.mean((z_q - lax.stop_gradient(z)) ** 2)
    z_q = z + lax.stop_gradient(z_q - z)
    z_q = z_q.transpose(0, 3, 1, 2)

    x = _conv_t(z_q, params['dec_w1'], params['dec_b1'], 1, 1)
    x = _res_stack(x, params['dec_res'])
    x = jax.nn.relu(_conv_t(x, params['dec_w2'], params['dec_b2'], 2, 1))
    x = _conv_t(x, params['dec_w3'], params['dec_b3'], 2, 1)
    return x, codebook_loss, commitment_loss, idx[:, None]


# speedup vs baseline: 3.9468x; 1.0898x over previous
import jax
import jax.numpy as jnp
from jax import lax
from jax.experimental import pallas as pl

EMBED_DIM = 64
NUM_EMB = 8192
H_DIM = 128
RES_H = 32

N_ROWS = 4 * 56 * 56
RB = 256
NB = N_ROWS // RB


def _conv(x, w, b, stride, padding):
    out = lax.conv_general_dilated(
        x, w, (stride, stride), [(padding, padding), (padding, padding)],
        dimension_numbers=('NCHW', 'OIHW', 'NCHW'))
    if b is not None:
        out = out + b[None, :, None, None]
    return out


def _conv_t(x, w, b, stride, padding):
    k = w.shape[2]
    pad = k - 1 - padding
    w2 = jnp.flip(w, (2, 3)).transpose(1, 0, 2, 3)
    out = lax.conv_general_dilated(
        x, w2, (1, 1), [(pad, pad), (pad, pad)], lhs_dilation=(stride, stride),
        dimension_numbers=('NCHW', 'OIHW', 'NCHW'))
    if b is not None:
        out = out + b[None, :, None, None]
    return out


def _res_stack(x, layers):
    for (w1, w2) in layers:
        h = _conv(jax.nn.relu(x), w1, None, 1, 1)
        h = _conv(jax.nn.relu(h), w2, None, 1, 0)
        x = x + h
    return jax.nn.relu(x)


def _lookup_body(idx_ref, emb_ref, out_ref):
    idxv = idx_ref[0, 0, :]
    cols = lax.broadcasted_iota(jnp.int32, (RB, NUM_EMB), 1)
    onehot = (idxv[:, None] == cols).astype(jnp.bfloat16)
    emb = emb_ref[...]
    eh = emb.astype(jnp.bfloat16)
    el = (emb - eh.astype(jnp.float32)).astype(jnp.bfloat16)
    dn = (((1,), (0,)), ((), ()))
    zq = lax.dot_general(
        onehot, jnp.concatenate([eh, el], axis=1), dn,
        preferred_element_type=jnp.float32)
    out_ref[...] = zq[:, :EMBED_DIM] + zq[:, EMBED_DIM:]


_lookup_call = pl.pallas_call(
    _lookup_body,
    grid=(NB,),
    in_specs=[
        pl.BlockSpec((1, 1, RB), lambda i: (i, 0, 0)),
        pl.BlockSpec((NUM_EMB, EMBED_DIM), lambda i: (0, 0)),
    ],
    out_specs=pl.BlockSpec((RB, EMBED_DIM), lambda i: (i, 0)),
    out_shape=jax.ShapeDtypeStruct((N_ROWS, EMBED_DIM), jnp.float32),
)


def kernel(patch, params):
    z = jax.nn.relu(_conv(patch, params['enc_w1'], params['enc_b1'], 2, 1))
    z = jax.nn.relu(_conv(z, params['enc_w2'], params['enc_b2'], 2, 1))
    z = _conv(z, params['enc_w3'], params['enc_b3'], 1, 1)
    z = _res_stack(z, params['enc_res'])
    z = _conv(z, params['preq_w'], params['preq_b'], 1, 0)
    z = z.transpose(0, 2, 3, 1)
    zf = z.reshape(-1, EMBED_DIM)
    emb = params['emb']

    dists = (jnp.sum(zf ** 2, axis=1, keepdims=True) + jnp.sum(emb ** 2, axis=1)
             - 2.0 * (zf @ emb.T))
    idx = jnp.argmin(dists, axis=1)

    z_q = _lookup_call(idx.reshape(NB, 1, RB), emb).reshape(z.shape)

    commitment_loss = jnp.mean((lax.stop_gradient(z_q) - z) ** 2)
    codebook_loss = jnp.mean((z_q - lax.stop_gradient(z)) ** 2)
    z_q = z + lax.stop_gradient(z_q - z)
    z_q = z_q.transpose(0, 3, 1, 2)

    x = _conv_t(z_q, params['dec_w1'], params['dec_b1'], 1, 1)
    x = _res_stack(x, params['dec_res'])
    x = jax.nn.relu(_conv_t(x, params['dec_w2'], params['dec_b2'], 2, 1))
    x = _conv_t(x, params['dec_w3'], params['dec_b3'], 2, 1)
    return x, codebook_loss, commitment_loss, idx[:, None]


# speedup vs baseline: 3.9934x; 1.0118x over previous
import jax
import jax.numpy as jnp
from jax import lax
from jax.experimental import pallas as pl

EMBED_DIM = 64
NUM_EMB = 8192
H_DIM = 128
RES_H = 32

N_ROWS = 4 * 56 * 56
RB = 448
NB = N_ROWS // RB


def _conv(x, w, b, stride, padding):
    out = lax.conv_general_dilated(
        x, w, (stride, stride), [(padding, padding), (padding, padding)],
        dimension_numbers=('NCHW', 'OIHW', 'NCHW'))
    if b is not None:
        out = out + b[None, :, None, None]
    return out


def _conv_t(x, w, b, stride, padding):
    k = w.shape[2]
    pad = k - 1 - padding
    w2 = jnp.flip(w, (2, 3)).transpose(1, 0, 2, 3)
    out = lax.conv_general_dilated(
        x, w2, (1, 1), [(pad, pad), (pad, pad)], lhs_dilation=(stride, stride),
        dimension_numbers=('NCHW', 'OIHW', 'NCHW'))
    if b is not None:
        out = out + b[None, :, None, None]
    return out


def _res_stack(x, layers):
    for (w1, w2) in layers:
        h = _conv(jax.nn.relu(x), w1, None, 1, 1)
        h = _conv(jax.nn.relu(h), w2, None, 1, 0)
        x = x + h
    return jax.nn.relu(x)


def _lookup_body(idx_ref, emb_ref, out_ref):
    idxv = idx_ref[0, 0, :]
    cols = lax.broadcasted_iota(jnp.int32, (RB, NUM_EMB), 1)
    onehot = (idxv[:, None] == cols).astype(jnp.bfloat16)
    emb = emb_ref[...]
    eh = emb.astype(jnp.bfloat16)
    el = (emb - eh.astype(jnp.float32)).astype(jnp.bfloat16)
    dn = (((1,), (0,)), ((), ()))
    zq = lax.dot_general(
        onehot, jnp.concatenate([eh, el], axis=1), dn,
        preferred_element_type=jnp.float32)
    out_ref[...] = zq[:, :EMBED_DIM] + zq[:, EMBED_DIM:]


_lookup_call = pl.pallas_call(
    _lookup_body,
    grid=(NB,),
    in_specs=[
        pl.BlockSpec((1, 1, RB), lambda i: (i, 0, 0)),
        pl.BlockSpec((NUM_EMB, EMBED_DIM), lambda i: (0, 0)),
    ],
    out_specs=pl.BlockSpec((RB, EMBED_DIM), lambda i: (i, 0)),
    out_shape=jax.ShapeDtypeStruct((N_ROWS, EMBED_DIM), jnp.float32),
)


def kernel(patch, params):
    z = jax.nn.relu(_conv(patch, params['enc_w1'], params['enc_b1'], 2, 1))
    z = jax.nn.relu(_conv(z, params['enc_w2'], params['enc_b2'], 2, 1))
    z = _conv(z, params['enc_w3'], params['enc_b3'], 1, 1)
    z = _res_stack(z, params['enc_res'])
    z = _conv(z, params['preq_w'], params['preq_b'], 1, 0)
    z = z.transpose(0, 2, 3, 1)
    zf = z.reshape(-1, EMBED_DIM)
    emb = params['emb']

    dists = (jnp.sum(zf ** 2, axis=1, keepdims=True) + jnp.sum(emb ** 2, axis=1)
             - 2.0 * (zf @ emb.T))
    idx = jnp.argmin(dists, axis=1)

    z_q = _lookup_call(idx.reshape(NB, 1, RB), emb).reshape(z.shape)

    commitment_loss = jnp.mean((lax.stop_gradient(z_q) - z) ** 2)
    codebook_loss = jnp.mean((z_q - lax.stop_gradient(z)) ** 2)
    z_q = z + lax.stop_gradient(z_q - z)
    z_q = z_q.transpose(0, 3, 1, 2)

    x = _conv_t(z_q, params['dec_w1'], params['dec_b1'], 1, 1)
    x = _res_stack(x, params['dec_res'])
    x = jax.nn.relu(_conv_t(x, params['dec_w2'], params['dec_b2'], 2, 1))
    x = _conv_t(x, params['dec_w3'], params['dec_b3'], 2, 1)
    return x, codebook_loss, commitment_loss, idx[:, None]
